# baseline (device time: 65034 ns/iter reference)
import jax
import jax.numpy as jnp
import numpy as np
from jax import lax
from jax.experimental import pallas as pl
from jax.experimental.pallas import tpu as pltpu

N_DEV = 8

_NX = np.array([1, 0, 3, 2, 5, 4, 7, 6])
_NY = np.array([3, 2, 1, 0, 7, 6, 5, 4])
_NZ = np.array([4, 5, 6, 7, 0, 1, 2, 3])

_ORDERS = [(_NX, _NY, _NZ), (_NY, _NZ, _NX), (_NZ, _NX, _NY)]
_ROWS = [88, 88, 80]
_OFFS = [0, 88, 176]
N_STREAM = 3
N_MSG = 7


def _lut(table, idx):
    r = jnp.int32(int(table[0]))
    for j in range(1, N_DEV):
        r = jnp.where(idx == j, jnp.int32(int(table[j])), r)
    return r


def _gelu(y):
    c = 0.7978845608028654
    return 0.5 * y * (1.0 + jnp.tanh(c * (y + 0.044715 * y * y * y)))


def kernel(x, w_mat):
    m_per, k = x.shape
    _, n_per = w_mat.shape

    def body(x_ref, w_ref, out_ref, xg, ssem, rsem):
        my = lax.axis_index("i")

        nx = _lut(_NX, my)
        ny = _lut(_NY, my)
        nz = _lut(_NZ, my)
        barrier_sem = pltpu.get_barrier_semaphore()
        for nbr in [nx, ny, nz]:
            pl.semaphore_signal(
                barrier_sem, inc=1,
                device_id=(nbr,), device_id_type=pl.DeviceIdType.MESH,
            )
        pl.semaphore_wait(barrier_sem, 3)

        w = w_ref[:, :]
        inflight = []

        def piece_ref(s, origin):
            return xg.at[pl.ds(origin * m_per + _OFFS[s], _ROWS[s]), :]

        def send(s, j, src, origin, dev):
            rdma = pltpu.make_async_remote_copy(
                src_ref=src, dst_ref=piece_ref(s, origin),
                send_sem=ssem.at[s, j], recv_sem=rsem.at[s, j],
                device_id=(dev,), device_id_type=pl.DeviceIdType.MESH,
            )
            rdma.start()
            inflight.append(rdma)

        def wait_recv(s, j, origin):
            pltpu.make_async_remote_copy(
                src_ref=piece_ref(s, origin), dst_ref=piece_ref(s, origin),
                send_sem=ssem.at[s, j], recv_sem=rsem.at[s, j],
                device_id=(my,), device_id_type=pl.DeviceIdType.MESH,
            ).wait_recv()

        def gemm_piece(s, origin):
            rows = pl.ds(origin * m_per + _OFFS[s], _ROWS[s])
            out_ref[rows, :] = _gelu(
                jnp.dot(xg[rows, :], w, preferred_element_type=jnp.float32)
            )

        nbrs, orig = [], []
        for s in range(N_STREAM):
            d1, d2, d3 = _ORDERS[s]
            nbrs.append((_lut(d1, my), _lut(d2, my), _lut(d3, my)))
            orig.append({
                "o1": _lut(d1, my), "o2": _lut(d2, my),
                "o12": _lut(d1[d2], my), "o3": _lut(d3, my),
                "o13": _lut(d1[d3], my), "o23": _lut(d2[d3], my),
                "o123": _lut(d1[d2[d3]], my),
            })

        def own_src(s):
            return x_ref.at[pl.ds(_OFFS[s], _ROWS[s]), :]

        for s in range(N_STREAM):
            send(s, 0, own_src(s), my, nbrs[s][0])
        for s in range(N_STREAM):
            send(s, 1, own_src(s), my, nbrs[s][1])
        for s in range(N_STREAM):
            send(s, 2, own_src(s), my, nbrs[s][2])

        out_ref[pl.ds(my * m_per, m_per), :] = _gelu(
            jnp.dot(x_ref[:, :], w, preferred_element_type=jnp.float32)
        )

        for s in range(N_STREAM):
            o = orig[s]
            wait_recv(s, 0, o["o1"])
            send(s, 3, piece_ref(s, o["o1"]), o["o1"], nbrs[s][1])
            send(s, 4, piece_ref(s, o["o1"]), o["o1"], nbrs[s][2])
        for s in range(N_STREAM):
            gemm_piece(s, orig[s]["o1"])

        for s in range(N_STREAM):
            o = orig[s]
            wait_recv(s, 1, o["o2"])
            send(s, 5, piece_ref(s, o["o2"]), o["o2"], nbrs[s][2])
        for s in range(N_STREAM):
            o = orig[s]
            wait_recv(s, 3, o["o12"])
            send(s, 6, piece_ref(s, o["o12"]), o["o12"], nbrs[s][2])
        for s in range(N_STREAM):
            gemm_piece(s, orig[s]["o2"])
            gemm_piece(s, orig[s]["o12"])

        for key, j in [("o3", 2), ("o13", 4), ("o23", 5), ("o123", 6)]:
            for s in range(N_STREAM):
                wait_recv(s, j, orig[s][key])
                gemm_piece(s, orig[s][key])

        for rdma in inflight:
            rdma.wait_send()

    return pl.pallas_call(
        body,
        out_shape=jax.ShapeDtypeStruct((N_DEV * m_per, n_per), jnp.float32),
        in_specs=[
            pl.BlockSpec(memory_space=pltpu.VMEM),
            pl.BlockSpec(memory_space=pltpu.VMEM),
        ],
        out_specs=pl.BlockSpec(memory_space=pltpu.VMEM),
        scratch_shapes=[
            pltpu.VMEM((N_DEV * m_per, k), jnp.float32),
            pltpu.SemaphoreType.DMA((N_STREAM, N_MSG)),
            pltpu.SemaphoreType.DMA((N_STREAM, N_MSG)),
        ],
        compiler_params=pltpu.CompilerParams(collective_id=0),
    )(x, w_mat)


# device time: 38980 ns/iter; 1.6684x vs baseline; 1.6684x over previous
import jax
import jax.numpy as jnp
import numpy as np
from jax import lax
from jax.experimental import pallas as pl
from jax.experimental.pallas import tpu as pltpu

N_DEV = 8

_NX = np.array([1, 0, 3, 2, 5, 4, 7, 6])
_NY = np.array([3, 2, 1, 0, 7, 6, 5, 4])
_NZ = np.array([4, 5, 6, 7, 0, 1, 2, 3])

_ORDERS = [(_NX, _NY, _NZ), (_NY, _NZ, _NX), (_NZ, _NX, _NY)]
_ROWS = [96, 80, 80]
_OFFS = [0, 96, 176]
N_STREAM = 3
N_MSG = 7


def _lut(table, idx):
    r = jnp.int32(int(table[0]))
    for j in range(1, N_DEV):
        r = jnp.where(idx == j, jnp.int32(int(table[j])), r)
    return r


def _gelu(y):
    c = 0.7978845608028654
    return 0.5 * y * (1.0 + jnp.tanh(c * (y + 0.044715 * y * y * y)))


def kernel(x, w_mat):
    m_per, k = x.shape
    _, n_per = w_mat.shape

    def body(x_ref, w_ref, out_ref, xg, xb, ssem, rsem):
        my = lax.axis_index("i")

        nx = _lut(_NX, my)
        ny = _lut(_NY, my)
        nz = _lut(_NZ, my)
        barrier_sem = pltpu.get_barrier_semaphore()
        for nbr in [nx, ny, nz]:
            pl.semaphore_signal(
                barrier_sem, inc=1,
                device_id=(nbr,), device_id_type=pl.DeviceIdType.MESH,
            )
        xb[:, :] = x_ref[:, :].astype(jnp.bfloat16)
        pl.semaphore_wait(barrier_sem, 3)

        w = w_ref[:, :]
        inflight = []

        def piece_ref(s, origin):
            return xg.at[pl.ds(origin * m_per + _OFFS[s], _ROWS[s]), :]

        def send(s, j, src, origin, dev):
            rdma = pltpu.make_async_remote_copy(
                src_ref=src, dst_ref=piece_ref(s, origin),
                send_sem=ssem.at[s, j], recv_sem=rsem.at[s, j],
                device_id=(dev,), device_id_type=pl.DeviceIdType.MESH,
            )
            rdma.start()
            inflight.append(rdma)

        def wait_recv(s, j, origin):
            pltpu.make_async_remote_copy(
                src_ref=piece_ref(s, origin), dst_ref=piece_ref(s, origin),
                send_sem=ssem.at[s, j], recv_sem=rsem.at[s, j],
                device_id=(my,), device_id_type=pl.DeviceIdType.MESH,
            ).wait_recv()

        def gemm_piece(s, origin):
            rows = pl.ds(origin * m_per + _OFFS[s], _ROWS[s])
            out_ref[rows, :] = _gelu(
                jnp.dot(xg[rows, :], w, preferred_element_type=jnp.float32)
            )

        nbrs, orig = [], []
        for s in range(N_STREAM):
            d1, d2, d3 = _ORDERS[s]
            nbrs.append((_lut(d1, my), _lut(d2, my), _lut(d3, my)))
            orig.append({
                "o1": _lut(d1, my), "o2": _lut(d2, my),
                "o12": _lut(d1[d2], my), "o3": _lut(d3, my),
                "o13": _lut(d1[d3], my), "o23": _lut(d2[d3], my),
                "o123": _lut(d1[d2[d3]], my),
            })

        def own_src(s):
            return xb.at[pl.ds(_OFFS[s], _ROWS[s]), :]

        for s in range(N_STREAM):
            send(s, 0, own_src(s), my, nbrs[s][0])
        for s in range(N_STREAM):
            send(s, 1, own_src(s), my, nbrs[s][1])
        for s in range(N_STREAM):
            send(s, 2, own_src(s), my, nbrs[s][2])

        out_ref[pl.ds(my * m_per, m_per), :] = _gelu(
            jnp.dot(x_ref[:, :], w, preferred_element_type=jnp.float32)
        )

        for s in range(N_STREAM):
            o = orig[s]
            wait_recv(s, 0, o["o1"])
            send(s, 3, piece_ref(s, o["o1"]), o["o1"], nbrs[s][1])
            send(s, 4, piece_ref(s, o["o1"]), o["o1"], nbrs[s][2])
        for s in range(N_STREAM):
            gemm_piece(s, orig[s]["o1"])

        for s in range(N_STREAM):
            o = orig[s]
            wait_recv(s, 1, o["o2"])
            send(s, 5, piece_ref(s, o["o2"]), o["o2"], nbrs[s][2])
        for s in range(N_STREAM):
            o = orig[s]
            wait_recv(s, 3, o["o12"])
            send(s, 6, piece_ref(s, o["o12"]), o["o12"], nbrs[s][2])
        for s in range(N_STREAM):
            gemm_piece(s, orig[s]["o2"])
            gemm_piece(s, orig[s]["o12"])

        for key, j in [("o3", 2), ("o13", 4), ("o23", 5), ("o123", 6)]:
            for s in range(N_STREAM):
                wait_recv(s, j, orig[s][key])
                gemm_piece(s, orig[s][key])

        for rdma in inflight:
            rdma.wait_send()

    return pl.pallas_call(
        body,
        out_shape=jax.ShapeDtypeStruct((N_DEV * m_per, n_per), jnp.float32),
        in_specs=[
            pl.BlockSpec(memory_space=pltpu.VMEM),
            pl.BlockSpec(memory_space=pltpu.VMEM),
        ],
        out_specs=pl.BlockSpec(memory_space=pltpu.VMEM),
        scratch_shapes=[
            pltpu.VMEM((N_DEV * m_per, k), jnp.bfloat16),
            pltpu.VMEM((m_per, k), jnp.bfloat16),
            pltpu.SemaphoreType.DMA((N_STREAM, N_MSG)),
            pltpu.SemaphoreType.DMA((N_STREAM, N_MSG)),
        ],
        compiler_params=pltpu.CompilerParams(collective_id=0),
    )(x, w_mat)


# device time: 38858 ns/iter; 1.6736x vs baseline; 1.0031x over previous
import jax
import jax.numpy as jnp
import numpy as np
from jax import lax
from jax.experimental import pallas as pl
from jax.experimental.pallas import tpu as pltpu

N_DEV = 8

_NX = np.array([1, 0, 3, 2, 5, 4, 7, 6])
_NY = np.array([3, 2, 1, 0, 7, 6, 5, 4])
_NZ = np.array([4, 5, 6, 7, 0, 1, 2, 3])

_ORDERS = [(_NX, _NY, _NZ), (_NY, _NZ, _NX), (_NZ, _NX, _NY)]
_ROWS = [96, 80, 80]
_OFFS = [0, 96, 176]
N_STREAM = 3
N_MSG = 7


def _lut(table, idx):
    r = jnp.int32(int(table[0]))
    for j in range(1, N_DEV):
        r = jnp.where(idx == j, jnp.int32(int(table[j])), r)
    return r


def _gelu(y):
    c = 0.7978845608028654
    return 0.5 * y * (1.0 + jnp.tanh(c * (y + 0.044715 * y * y * y)))


def kernel(x, w_mat):
    m_per, k = x.shape
    _, n_per = w_mat.shape

    def body(x_ref, w_ref, out_ref, xg, xb, wb, ssem, rsem):
        my = lax.axis_index("i")

        nx = _lut(_NX, my)
        ny = _lut(_NY, my)
        nz = _lut(_NZ, my)
        barrier_sem = pltpu.get_barrier_semaphore()
        for nbr in [nx, ny, nz]:
            pl.semaphore_signal(
                barrier_sem, inc=1,
                device_id=(nbr,), device_id_type=pl.DeviceIdType.MESH,
            )
        xb[:, :] = x_ref[:, :].astype(jnp.bfloat16)
        wb[:, :] = w_ref[:, :].astype(jnp.bfloat16)
        pl.semaphore_wait(barrier_sem, 3)

        w = wb[:, :]
        inflight = []

        def piece_ref(s, origin):
            return xg.at[pl.ds(origin * m_per + _OFFS[s], _ROWS[s]), :]

        def send(s, j, src, origin, dev):
            rdma = pltpu.make_async_remote_copy(
                src_ref=src, dst_ref=piece_ref(s, origin),
                send_sem=ssem.at[s, j], recv_sem=rsem.at[s, j],
                device_id=(dev,), device_id_type=pl.DeviceIdType.MESH,
            )
            rdma.start()
            inflight.append(rdma)

        def wait_recv(s, j, origin):
            pltpu.make_async_remote_copy(
                src_ref=piece_ref(s, origin), dst_ref=piece_ref(s, origin),
                send_sem=ssem.at[s, j], recv_sem=rsem.at[s, j],
                device_id=(my,), device_id_type=pl.DeviceIdType.MESH,
            ).wait_recv()

        def gemm_piece(s, origin):
            rows = pl.ds(origin * m_per + _OFFS[s], _ROWS[s])
            out_ref[rows, :] = _gelu(
                jnp.dot(xg[rows, :], w, preferred_element_type=jnp.float32)
            )

        nbrs, orig = [], []
        for s in range(N_STREAM):
            d1, d2, d3 = _ORDERS[s]
            nbrs.append((_lut(d1, my), _lut(d2, my), _lut(d3, my)))
            orig.append({
                "o1": _lut(d1, my), "o2": _lut(d2, my),
                "o12": _lut(d1[d2], my), "o3": _lut(d3, my),
                "o13": _lut(d1[d3], my), "o23": _lut(d2[d3], my),
                "o123": _lut(d1[d2[d3]], my),
            })

        def own_src(s):
            return xb.at[pl.ds(_OFFS[s], _ROWS[s]), :]

        for s in range(N_STREAM):
            send(s, 0, own_src(s), my, nbrs[s][0])
        for s in range(N_STREAM):
            send(s, 1, own_src(s), my, nbrs[s][1])
        for s in range(N_STREAM):
            send(s, 2, own_src(s), my, nbrs[s][2])

        out_ref[pl.ds(my * m_per, m_per), :] = _gelu(
            jnp.dot(xb[:, :], w, preferred_element_type=jnp.float32)
        )

        for s in range(N_STREAM):
            o = orig[s]
            wait_recv(s, 0, o["o1"])
            send(s, 3, piece_ref(s, o["o1"]), o["o1"], nbrs[s][1])
            send(s, 4, piece_ref(s, o["o1"]), o["o1"], nbrs[s][2])
        for s in range(N_STREAM):
            gemm_piece(s, orig[s]["o1"])

        for s in range(N_STREAM):
            o = orig[s]
            wait_recv(s, 1, o["o2"])
            send(s, 5, piece_ref(s, o["o2"]), o["o2"], nbrs[s][2])
        for s in range(N_STREAM):
            o = orig[s]
            wait_recv(s, 3, o["o12"])
            send(s, 6, piece_ref(s, o["o12"]), o["o12"], nbrs[s][2])
        for s in range(N_STREAM):
            gemm_piece(s, orig[s]["o2"])
            gemm_piece(s, orig[s]["o12"])

        for key, j in [("o3", 2), ("o13", 4), ("o23", 5), ("o123", 6)]:
            for s in range(N_STREAM):
                wait_recv(s, j, orig[s][key])
                gemm_piece(s, orig[s][key])

        for rdma in inflight:
            rdma.wait_send()

    return pl.pallas_call(
        body,
        out_shape=jax.ShapeDtypeStruct((N_DEV * m_per, n_per), jnp.float32),
        in_specs=[
            pl.BlockSpec(memory_space=pltpu.VMEM),
            pl.BlockSpec(memory_space=pltpu.VMEM),
        ],
        out_specs=pl.BlockSpec(memory_space=pltpu.VMEM),
        scratch_shapes=[
            pltpu.VMEM((N_DEV * m_per, k), jnp.bfloat16),
            pltpu.VMEM((m_per, k), jnp.bfloat16),
            pltpu.VMEM((k, n_per), jnp.bfloat16),
            pltpu.SemaphoreType.DMA((N_STREAM, N_MSG)),
            pltpu.SemaphoreType.DMA((N_STREAM, N_MSG)),
        ],
        compiler_params=pltpu.CompilerParams(collective_id=0),
    )(x, w_mat)
